# Initial kernel scaffold; baseline (speedup 1.0000x reference)
import functools

import jax
import jax.numpy as jnp
from jax import lax
from jax.experimental import pallas as pl
from jax.experimental.pallas import tpu as pltpu
from jax.experimental.pallas import tpu_sc as plsc

N = 10000
E = 320000
D = 128

NC = 2
NS = 16
NW = NC * NS
CL = 16

CHUNK = 128
CPW = -(-E // (CHUNK * NW))
EP = CPW * NW * CHUNK
NPAD = 10240
RPT = NPAD // NS


def _make_sc_agg(with_counts: bool):
    mesh = plsc.VectorSubcoreMesh(core_axis_name="c", subcore_axis_name="s")
    out_type = [jax.ShapeDtypeStruct((NC, NPAD, D), jnp.float32)]
    scratch = [
        pltpu.VMEM((CHUNK,), jnp.int32),
        pltpu.VMEM((CHUNK,), jnp.int32),
        pltpu.VMEM((CHUNK, D), jnp.float32),
        pltpu.VMEM_SHARED((NPAD, D), jnp.float32),
        pltpu.SemaphoreType.DMA,
    ]
    if with_counts:
        out_type.append(jax.ShapeDtypeStruct((NC, NPAD, CL), jnp.float32))
        scratch += [
            pltpu.VMEM((CHUNK, CL), jnp.float32),
            pltpu.VMEM_SHARED((NPAD, CL), jnp.float32),
        ]

    def body(*refs):
        if with_counts:
            (x_hbm, src_hbm, dst_hbm, zbig_hbm, zsml_hbm, ones_hbm,
             out_agg, out_cnt, sidx, didx, rows, acc, sem, ones_v, cacc) = refs
        else:
            (x_hbm, src_hbm, dst_hbm, zbig_hbm,
             out_agg, sidx, didx, rows, acc, sem) = refs
        cid = lax.axis_index("c")
        sid = lax.axis_index("s")
        wid = sid * NC + cid

        pltpu.sync_copy(zbig_hbm, acc.at[pl.ds(sid * RPT, RPT)])
        if with_counts:
            pltpu.sync_copy(zsml_hbm, cacc.at[pl.ds(sid * RPT, RPT)])
            pltpu.sync_copy(ones_hbm, ones_v)
        plsc.subcore_barrier()

        base0 = wid * CPW * CHUNK

        def step(c, carry):
            b = base0 + c * CHUNK
            pltpu.sync_copy(src_hbm.at[pl.ds(b, CHUNK)], sidx)
            pltpu.sync_copy(dst_hbm.at[pl.ds(b, CHUNK)], didx)
            pltpu.async_copy(x_hbm.at[sidx], rows, sem).wait()
            pltpu.sync_copy(rows, acc.at[didx], add=True)
            if with_counts:
                pltpu.sync_copy(ones_v, cacc.at[didx], add=True)
            return carry

        lax.fori_loop(0, CPW, step, 0)
        plsc.subcore_barrier()

        sl = pl.ds(sid * RPT, RPT)
        pltpu.sync_copy(acc.at[sl], out_agg.at[cid].at[sl])
        if with_counts:
            pltpu.sync_copy(cacc.at[sl], out_cnt.at[cid].at[sl])

    return pl.kernel(
        body,
        out_type=tuple(out_type) if with_counts else out_type[0],
        mesh=mesh,
        scratch_types=scratch,
    )


_sc_agg_counts = _make_sc_agg(True)
_sc_agg = _make_sc_agg(False)


def _dense_body(residual, aggp, cntp, x, wlt, bl, wrt, out):
    c = cntp[...]
    cnt = c[0, :, 0:1] + c[1, :, 0:1]
    inv = 1.0 / jnp.maximum(cnt, 1.0)
    a = aggp[...]
    agg = (a[0] + a[1]) * inv
    h = jnp.dot(agg, wlt[...], preferred_element_type=jnp.float32)
    h = h + bl[...]
    h = h + jnp.dot(x[...], wrt[...], preferred_element_type=jnp.float32)
    h = jnp.maximum(h, 0.0)
    if residual:
        h = h + x[...]
    out[...] = h


BN = 1000


def _make_dense(residual: bool):
    return pl.pallas_call(
        functools.partial(_dense_body, residual),
        grid=(N // BN,),
        in_specs=[
            pl.BlockSpec((NC, BN, D), lambda i: (0, i, 0)),
            pl.BlockSpec((NC, BN, CL), lambda i: (0, i, 0)),
            pl.BlockSpec((BN, D), lambda i: (i, 0)),
            pl.BlockSpec((D, D), lambda i: (0, 0)),
            pl.BlockSpec((1, D), lambda i: (0, 0)),
            pl.BlockSpec((D, D), lambda i: (0, 0)),
        ],
        out_specs=pl.BlockSpec((BN, D), lambda i: (i, 0)),
        out_shape=jax.ShapeDtypeStruct((N, D), jnp.float32),
    )


_dense_first = _make_dense(False)
_dense_resid = _make_dense(True)


def kernel(x, edge_index, Wl1, bl1, Wr1, Wl2, bl2, Wr2, Wl3, bl3, Wr3,
           Wl4, bl4, Wr4):
    src = edge_index[0]
    dst = edge_index[1]
    pad_s = jnp.zeros((EP - E,), jnp.int32)
    pad_d = jnp.full((EP - E,), N, jnp.int32)
    srcp = jnp.concatenate([src, pad_s])
    dstp = jnp.concatenate([dst, pad_d])

    zbig = jnp.zeros((RPT, D), jnp.float32)
    zsml = jnp.zeros((RPT, CL), jnp.float32)
    ones = jnp.ones((CHUNK, CL), jnp.float32)

    aggp, cntp = _sc_agg_counts(x, srcp, dstp, zbig, zsml, ones)
    cur = _dense_first(aggp, cntp, x, Wl1.T, bl1[None, :], Wr1.T)
    for (Wl, bl, Wr) in ((Wl2, bl2, Wr2), (Wl3, bl3, Wr3), (Wl4, bl4, Wr4)):
        aggp = _sc_agg(cur, srcp, dstp, zbig)
        cur = _dense_resid(aggp, cntp, cur, Wl.T, bl[None, :], Wr.T)
    return cur


# speedup vs baseline: 3.2018x; 3.2018x over previous
import jax
import jax.numpy as jnp
from jax import lax
from jax.experimental import pallas as pl
from jax.experimental.pallas import tpu as pltpu
from jax.experimental.pallas import tpu_sc as plsc

N = 10000
E = 320000
D = 128

NC = 2
NS = 16
NW = NC * NS

CHUNK = 128
CPW = -(-E // (CHUNK * NW))
EP = CPW * NW * CHUNK
NPAD = 10240
RPT = NPAD // NS


def _make_sc_agg():
    mesh = plsc.VectorSubcoreMesh(core_axis_name="c", subcore_axis_name="s")

    def body(x_hbm, src_hbm, dst_hbm, zbig_hbm, out_agg,
             sidx, didx, rows, acc, sem):
        cid = lax.axis_index("c")
        sid = lax.axis_index("s")
        wid = sid * NC + cid

        pltpu.sync_copy(zbig_hbm, acc.at[pl.ds(sid * RPT, RPT)])
        plsc.subcore_barrier()

        base0 = wid * CPW * CHUNK

        def step(c, carry):
            b = base0 + c * CHUNK
            pltpu.sync_copy(src_hbm.at[pl.ds(b, CHUNK)], sidx)
            pltpu.sync_copy(dst_hbm.at[pl.ds(b, CHUNK)], didx)
            pltpu.async_copy(x_hbm.at[sidx], rows, sem).wait()
            pltpu.sync_copy(rows, acc.at[didx], add=True)
            return carry

        lax.fori_loop(0, CPW, step, 0)
        plsc.subcore_barrier()

        pltpu.sync_copy(acc.at[pl.ds(sid * RPT, RPT)],
                        out_agg.at[pl.ds(cid * NPAD + sid * RPT, RPT)])

    return pl.kernel(
        body,
        out_type=jax.ShapeDtypeStruct((NC * NPAD, D), jnp.float32),
        mesh=mesh,
        scratch_types=[
            pltpu.VMEM((CHUNK,), jnp.int32),
            pltpu.VMEM((CHUNK,), jnp.int32),
            pltpu.VMEM((CHUNK, D), jnp.float32),
            pltpu.VMEM_SHARED((NPAD, D), jnp.float32),
            pltpu.SemaphoreType.DMA,
        ],
    )


_sc_agg = _make_sc_agg()


def _dense_body_first(aggp, cntp, x, wlt, bl, wrt, out):
    _dense_common(False, aggp, cntp, x, wlt, bl, wrt, out)


def _dense_body_resid(aggp, cntp, x, wlt, bl, wrt, out):
    _dense_common(True, aggp, cntp, x, wlt, bl, wrt, out)


def _dense_common(residual, aggp, cntp, x, wlt, bl, wrt, out):
    c = cntp[...]
    cnt = c[0, :, 0:1] + c[1, :, 0:1]
    inv = 1.0 / jnp.maximum(cnt, 1.0)
    a = aggp[...]
    agg = (a[0] + a[1]) * inv
    h = jnp.dot(agg, wlt[...], preferred_element_type=jnp.float32)
    h = h + bl[...]
    h = h + jnp.dot(x[...], wrt[...], preferred_element_type=jnp.float32)
    h = jnp.maximum(h, 0.0)
    if residual:
        h = h + x[...]
    out[...] = h


BN = 1000


def _make_dense(residual: bool):
    return pl.pallas_call(
        _dense_body_resid if residual else _dense_body_first,
        grid=(N // BN,),
        in_specs=[
            pl.BlockSpec((NC, BN, D), lambda i: (0, i, 0)),
            pl.BlockSpec((NC, BN, D), lambda i: (0, i, 0)),
            pl.BlockSpec((BN, D), lambda i: (i, 0)),
            pl.BlockSpec((D, D), lambda i: (0, 0)),
            pl.BlockSpec((1, D), lambda i: (0, 0)),
            pl.BlockSpec((D, D), lambda i: (0, 0)),
        ],
        out_specs=pl.BlockSpec((BN, D), lambda i: (i, 0)),
        out_shape=jax.ShapeDtypeStruct((N, D), jnp.float32),
    )


_dense_first = _make_dense(False)
_dense_resid = _make_dense(True)


def kernel(x, edge_index, Wl1, bl1, Wr1, Wl2, bl2, Wr2, Wl3, bl3, Wr3,
           Wl4, bl4, Wr4):
    src = edge_index[0]
    dst = edge_index[1]
    pad_s = jnp.zeros((EP - E,), jnp.int32)
    pad_d = jnp.full((EP - E,), N, jnp.int32)
    srcp = jnp.concatenate([src, pad_s])
    dstp = jnp.concatenate([dst, pad_d])

    zbig = jnp.zeros((RPT, D), jnp.float32)
    ones_x = jnp.ones((N, D), jnp.float32)

    cntp = _sc_agg(ones_x, srcp, dstp, zbig).reshape(NC, NPAD, D)
    aggp = _sc_agg(x, srcp, dstp, zbig).reshape(NC, NPAD, D)
    cur = _dense_first(aggp, cntp, x, Wl1.T, bl1[None, :], Wr1.T)
    for (Wl, bl, Wr) in ((Wl2, bl2, Wr2), (Wl3, bl3, Wr3), (Wl4, bl4, Wr4)):
        aggp = _sc_agg(cur, srcp, dstp, zbig).reshape(NC, NPAD, D)
        cur = _dense_resid(aggp, cntp, cur, Wl.T, bl[None, :], Wr.T)
    return cur


# speedup vs baseline: 3.4093x; 1.0648x over previous
import jax
import jax.numpy as jnp
from jax import lax
from jax.experimental import pallas as pl
from jax.experimental.pallas import tpu as pltpu
from jax.experimental.pallas import tpu_sc as plsc

N = 10000
E = 320000
D = 128

NC = 2
NS = 16
NW = NC * NS

CHUNK = 128
NB = 2
CPW = 80
NG = CPW // NB
EP = CPW * NW * CHUNK
NPAD = 10240
RPT = NPAD // NS


def _make_sc_agg(with_gather: bool):
    mesh = plsc.VectorSubcoreMesh(core_axis_name="c", subcore_axis_name="s")
    scratch = [
        pltpu.VMEM((CPW, CHUNK), jnp.int32),
        pltpu.VMEM_SHARED((NPAD, D), jnp.float32),
    ]
    if with_gather:
        scratch += [
            [pltpu.VMEM((CHUNK,), jnp.int32)] * NB,
            [pltpu.VMEM((CHUNK, D), jnp.float32)] * NB,
            [pltpu.SemaphoreType.DMA] * NB,
            [pltpu.SemaphoreType.DMA] * NB,
        ]
    else:
        scratch += [pltpu.VMEM((CHUNK, D), jnp.float32)]

    def body(*refs):
        if with_gather:
            (x_hbm, src_hbm, dst_hbm, zbig_hbm, out_agg,
             didx, acc, sidx, rows, gsem, isem) = refs
        else:
            (ones_hbm, dst_hbm, zbig_hbm, out_agg, didx, acc, ones_v) = refs
        cid = lax.axis_index("c")
        sid = lax.axis_index("s")
        wid = sid * NC + cid

        pltpu.sync_copy(zbig_hbm, acc.at[pl.ds(sid * RPT, RPT)])
        pltpu.sync_copy(dst_hbm.at[pl.ds(wid * CPW, CPW)], didx)
        if with_gather:
            row0 = wid * CPW
            for b in range(NB):
                pltpu.async_copy(src_hbm.at[row0 + b], sidx[b], isem[b])
            for b in range(NB):
                pltpu.make_async_copy(src_hbm.at[0], sidx[b], isem[b]).wait()
                pltpu.async_copy(x_hbm.at[sidx[b]], rows[b], gsem[b])
        else:
            pltpu.sync_copy(ones_hbm, ones_v)
        plsc.subcore_barrier()

        if with_gather:
            def step(g, carry):
                for b in range(NB):
                    c = g * NB + b
                    pltpu.make_async_copy(x_hbm.at[sidx[b]], rows[b],
                                          gsem[b]).wait()

                    @pl.when(g < NG - 1)
                    def _():
                        pltpu.async_copy(src_hbm.at[row0 + c + NB],
                                         sidx[b], isem[b])

                    pltpu.sync_copy(rows[b], acc.at[didx.at[c]], add=True)

                    @pl.when(g < NG - 1)
                    def _():
                        pltpu.make_async_copy(src_hbm.at[0], sidx[b],
                                              isem[b]).wait()
                        pltpu.async_copy(x_hbm.at[sidx[b]], rows[b], gsem[b])
                return carry

            lax.fori_loop(0, NG, step, 0)
        else:
            def step(c, carry):
                pltpu.sync_copy(ones_v, acc.at[didx.at[c]], add=True)
                return carry

            lax.fori_loop(0, CPW, step, 0)
        plsc.subcore_barrier()

        pltpu.sync_copy(acc.at[pl.ds(sid * RPT, RPT)],
                        out_agg.at[pl.ds(cid * NPAD + sid * RPT, RPT)])

    return pl.kernel(
        body,
        out_type=jax.ShapeDtypeStruct((NC * NPAD, D), jnp.float32),
        mesh=mesh,
        scratch_types=scratch,
    )


_sc_agg = _make_sc_agg(True)
_sc_cnt = _make_sc_agg(False)


def _dense_body_first(aggp, cntp, x, wlt, bl, wrt, out):
    _dense_common(False, aggp, cntp, x, wlt, bl, wrt, out)


def _dense_body_resid(aggp, cntp, x, wlt, bl, wrt, out):
    _dense_common(True, aggp, cntp, x, wlt, bl, wrt, out)


def _dense_common(residual, aggp, cntp, x, wlt, bl, wrt, out):
    c = cntp[...]
    cnt = c[0, :, 0:1] + c[1, :, 0:1]
    inv = 1.0 / jnp.maximum(cnt, 1.0)
    a = aggp[...]
    agg = (a[0] + a[1]) * inv
    h = jnp.dot(agg, wlt[...], preferred_element_type=jnp.float32)
    h = h + bl[...]
    h = h + jnp.dot(x[...], wrt[...], preferred_element_type=jnp.float32)
    h = jnp.maximum(h, 0.0)
    if residual:
        h = h + x[...]
    out[...] = h


BN = 1000


def _make_dense(residual: bool):
    return pl.pallas_call(
        _dense_body_resid if residual else _dense_body_first,
        grid=(N // BN,),
        in_specs=[
            pl.BlockSpec((NC, BN, D), lambda i: (0, i, 0)),
            pl.BlockSpec((NC, BN, D), lambda i: (0, i, 0)),
            pl.BlockSpec((BN, D), lambda i: (i, 0)),
            pl.BlockSpec((D, D), lambda i: (0, 0)),
            pl.BlockSpec((1, D), lambda i: (0, 0)),
            pl.BlockSpec((D, D), lambda i: (0, 0)),
        ],
        out_specs=pl.BlockSpec((BN, D), lambda i: (i, 0)),
        out_shape=jax.ShapeDtypeStruct((N, D), jnp.float32),
    )


_dense_first = _make_dense(False)
_dense_resid = _make_dense(True)


def kernel(x, edge_index, Wl1, bl1, Wr1, Wl2, bl2, Wr2, Wl3, bl3, Wr3,
           Wl4, bl4, Wr4):
    src = edge_index[0]
    dst = edge_index[1]
    pad_s = jnp.zeros((EP - E,), jnp.int32)
    pad_d = jnp.full((EP - E,), N, jnp.int32)
    srcp = jnp.concatenate([src, pad_s]).reshape(NW * CPW, CHUNK)
    dstp = jnp.concatenate([dst, pad_d]).reshape(NW * CPW, CHUNK)

    zbig = jnp.zeros((RPT, D), jnp.float32)
    ones_b = jnp.ones((CHUNK, D), jnp.float32)

    cntp = _sc_cnt(ones_b, dstp, zbig).reshape(NC, NPAD, D)
    aggp = _sc_agg(x, srcp, dstp, zbig).reshape(NC, NPAD, D)
    cur = _dense_first(aggp, cntp, x, Wl1.T, bl1[None, :], Wr1.T)
    for (Wl, bl, Wr) in ((Wl2, bl2, Wr2), (Wl3, bl3, Wr3), (Wl4, bl4, Wr4)):
        aggp = _sc_agg(cur, srcp, dstp, zbig).reshape(NC, NPAD, D)
        cur = _dense_resid(aggp, cntp, cur, Wl.T, bl[None, :], Wr.T)
    return cur


# speedup vs baseline: 11.5876x; 3.3988x over previous
import jax
import jax.numpy as jnp
from jax import lax
from jax.experimental import pallas as pl
from jax.experimental.pallas import tpu as pltpu
from jax.experimental.pallas import tpu_sc as plsc

N = 10000
E = 320000
D = 128

NC = 2
NS = 16
NW = NC * NS

CHUNK = 128
NB = 3
ND = 2 * NB
CPW = 84
CROW = 88
NG = CPW // ND
EP = CPW * NW * CHUNK
NPAD = 10016
RPT = 632
RPT_LAST = NPAD - 15 * RPT


def _make_sc_agg(with_gather: bool):
    mesh = plsc.VectorSubcoreMesh(core_axis_name="c", subcore_axis_name="s")
    scratch = [
        pltpu.VMEM_SHARED((NPAD, D), jnp.float32),
    ]
    if with_gather:
        scratch += [
            [pltpu.VMEM((CHUNK,), jnp.int32)] * NB,
            [pltpu.VMEM((CHUNK,), jnp.int32)] * ND,
            [pltpu.VMEM((CHUNK, D), jnp.float32)] * NB,
            [pltpu.SemaphoreType.DMA] * NB,
            [pltpu.SemaphoreType.DMA] * NB,
            [pltpu.SemaphoreType.DMA] * NB,
        ]
    else:
        scratch += [
            pltpu.VMEM((CROW, CHUNK), jnp.int32),
            pltpu.VMEM((CHUNK, D), jnp.float32),
        ]

    def body(*refs):
        if with_gather:
            (x_hbm, src_hbm, dst_hbm, zbig_hbm, out_agg,
             acc, sidx, didx, rows, gsem, ssem, isem) = refs
        else:
            (ones_hbm, dst_hbm, zbig_hbm, out_agg, acc, didx2, ones_v) = refs
        cid = lax.axis_index("c")
        sid = lax.axis_index("s")
        wid = sid * NC + cid
        row0 = wid * CROW

        @pl.when(sid < NS - 1)
        def _():
            pltpu.sync_copy(zbig_hbm, acc.at[pl.ds(sid * RPT, RPT)])

        @pl.when(sid == NS - 1)
        def _():
            pltpu.sync_copy(zbig_hbm.at[pl.ds(0, RPT_LAST)],
                            acc.at[pl.ds((NS - 1) * RPT, RPT_LAST)])

        if with_gather:
            for c in range(NB):
                pltpu.async_copy(src_hbm.at[row0 + c], sidx[c], isem[c])
                pltpu.async_copy(dst_hbm.at[row0 + c], didx[c], isem[c])
            for c in range(NB - 1):
                pltpu.make_async_copy(src_hbm.at[0], sidx[c], isem[c]).wait()
                pltpu.make_async_copy(dst_hbm.at[0], didx[c], isem[c]).wait()
                pltpu.async_copy(x_hbm.at[sidx[c]], rows[c], gsem[c])
        else:
            pltpu.sync_copy(dst_hbm.at[pl.ds(row0, CROW)], didx2)
            pltpu.sync_copy(ones_hbm, ones_v)
        plsc.subcore_barrier()

        if with_gather:
            def step(g, carry):
                for j in range(ND):
                    c = g * ND + j
                    s0 = j % NB
                    s1 = (j + 2) % NB
                    d0 = j
                    d3 = (j + 3) % ND
                    pltpu.make_async_copy(x_hbm.at[sidx[s0]], rows[s0],
                                          gsem[s0]).wait()

                    @pl.when(c + 3 < CPW)
                    def _():
                        pltpu.async_copy(src_hbm.at[row0 + c + 3],
                                         sidx[s0], isem[s0])
                        pltpu.async_copy(dst_hbm.at[row0 + c + 3],
                                         didx[d3], isem[s0])

                    pltpu.async_copy(rows[s0], acc.at[didx[d0]], ssem[s0],
                                     add=True)

                    @pl.when(c >= 1)
                    def _():
                        pltpu.make_async_copy(rows[s1], acc.at[didx[d0]],
                                              ssem[s1]).wait()

                    @pl.when(c + 2 < CPW)
                    def _():
                        pltpu.make_async_copy(src_hbm.at[0], sidx[s1],
                                              isem[s1]).wait()
                        pltpu.make_async_copy(dst_hbm.at[0], didx[(j + 2) % ND],
                                              isem[s1]).wait()
                        pltpu.async_copy(x_hbm.at[sidx[s1]], rows[s1],
                                         gsem[s1])
                return carry

            lax.fori_loop(0, NG, step, 0)
            sl = (CPW - 1) % NB
            pltpu.make_async_copy(rows[sl], acc.at[didx[0]], ssem[sl]).wait()
        else:
            def step(c, carry):
                pltpu.sync_copy(ones_v, acc.at[didx2.at[c]], add=True)
                return carry

            lax.fori_loop(0, CPW, step, 0)
        plsc.subcore_barrier()

        @pl.when(sid < NS - 1)
        def _():
            pltpu.sync_copy(acc.at[pl.ds(sid * RPT, RPT)],
                            out_agg.at[pl.ds(cid * NPAD + sid * RPT, RPT)])

        @pl.when(sid == NS - 1)
        def _():
            pltpu.sync_copy(
                acc.at[pl.ds((NS - 1) * RPT, RPT_LAST)],
                out_agg.at[pl.ds(cid * NPAD + (NS - 1) * RPT, RPT_LAST)])

    return pl.kernel(
        body,
        out_type=jax.ShapeDtypeStruct((NC * NPAD, D), jnp.float32),
        mesh=mesh,
        scratch_types=scratch,
    )


_sc_agg = _make_sc_agg(True)
_sc_cnt = _make_sc_agg(False)


def _dense_body_first(aggp, cntp, x, wlt, bl, wrt, out):
    _dense_common(False, aggp, cntp, x, wlt, bl, wrt, out)


def _dense_body_resid(aggp, cntp, x, wlt, bl, wrt, out):
    _dense_common(True, aggp, cntp, x, wlt, bl, wrt, out)


def _dense_common(residual, aggp, cntp, x, wlt, bl, wrt, out):
    c = cntp[...]
    cnt = c[0, :, 0:1] + c[1, :, 0:1]
    inv = 1.0 / jnp.maximum(cnt, 1.0)
    a = aggp[...]
    agg = (a[0] + a[1]) * inv
    h = jnp.dot(agg, wlt[...], preferred_element_type=jnp.float32)
    h = h + bl[...]
    h = h + jnp.dot(x[...], wrt[...], preferred_element_type=jnp.float32)
    h = jnp.maximum(h, 0.0)
    if residual:
        h = h + x[...]
    out[...] = h


BN = 1000


def _make_dense(residual: bool):
    return pl.pallas_call(
        _dense_body_resid if residual else _dense_body_first,
        grid=(N // BN,),
        in_specs=[
            pl.BlockSpec((NC, BN, D), lambda i: (0, i, 0)),
            pl.BlockSpec((NC, BN, D), lambda i: (0, i, 0)),
            pl.BlockSpec((BN, D), lambda i: (i, 0)),
            pl.BlockSpec((D, D), lambda i: (0, 0)),
            pl.BlockSpec((1, D), lambda i: (0, 0)),
            pl.BlockSpec((D, D), lambda i: (0, 0)),
        ],
        out_specs=pl.BlockSpec((BN, D), lambda i: (i, 0)),
        out_shape=jax.ShapeDtypeStruct((N, D), jnp.float32),
    )


_dense_first = _make_dense(False)
_dense_resid = _make_dense(True)


def kernel(x, edge_index, Wl1, bl1, Wr1, Wl2, bl2, Wr2, Wl3, bl3, Wr3,
           Wl4, bl4, Wr4):
    src = edge_index[0]
    dst = edge_index[1]
    npad = EP - E
    pad_s = (jnp.arange(npad, dtype=jnp.int32) * 61) % N
    pad_d = N + (jnp.arange(npad, dtype=jnp.int32) % (NPAD - N))
    srcw = jnp.concatenate([src, pad_s]).reshape(NW, CPW * CHUNK)
    dstw = jnp.concatenate([dst, pad_d]).reshape(NW, CPW * CHUNK)
    fill = (CROW - CPW) * CHUNK
    srcp = jnp.concatenate(
        [srcw, jnp.zeros((NW, fill), jnp.int32)], axis=1
    ).reshape(NW * CROW, CHUNK)
    dstp = jnp.concatenate(
        [dstw, jnp.full((NW, fill), N, jnp.int32)], axis=1
    ).reshape(NW * CROW, CHUNK)

    zbig = jnp.zeros((RPT, D), jnp.float32)
    ones_b = jnp.ones((CHUNK, D), jnp.float32)

    cntp = _sc_cnt(ones_b, dstp, zbig).reshape(NC, NPAD, D)
    aggp = _sc_agg(x, srcp, dstp, zbig).reshape(NC, NPAD, D)
    cur = _dense_first(aggp, cntp, x, Wl1.T, bl1[None, :], Wr1.T)
    for (Wl, bl, Wr) in ((Wl2, bl2, Wr2), (Wl3, bl3, Wr3), (Wl4, bl4, Wr4)):
        aggp = _sc_agg(cur, srcp, dstp, zbig).reshape(NC, NPAD, D)
        cur = _dense_resid(aggp, cntp, cur, Wl.T, bl[None, :], Wr.T)
    return cur
